# Initial kernel scaffold; baseline (speedup 1.0000x reference)
import jax
import jax.numpy as jnp
from jax.experimental import pallas as pl


def kernel(x, edge_index, edge_weight, batch, W_gcn, b_gcn, g1_W, g1_as, g1_ad, g1_b, g2_W, g2_as, g2_ad, g2_b, fc1_W, fc1_b, fc2_W, fc2_b, pro_bias, gfc1_W, gfc1_b, gfc2_W, gfc2_b):
    raise NotImplementedError("write your pallas kernel here")


# speedup vs baseline: 1.1199x; 1.1199x over previous
import functools

import jax
import jax.numpy as jnp
from jax.experimental import pallas as pl
from jax.experimental.pallas import tpu as pltpu

N = 50000
E = 800000
F_IN = 16
HID = 64
HEADS = 4
GRAPHS = 256
OUT_DIM = 128


def _mlp_head_body(pooled_parts_ref, w1_ref, b1_ref, w2_ref, b2_ref, out_ref):
    pooled = jnp.max(pooled_parts_ref[...], axis=0)
    pooled = jnp.where(jnp.isfinite(pooled), pooled, 0.0)
    h = jax.nn.relu(pooled @ w1_ref[...] + b1_ref[...])
    out_ref[...] = h @ w2_ref[...] + b2_ref[...]


def _mlp_head(pooled_parts, w1, b1, w2, b2):
    return pl.pallas_call(
        _mlp_head_body,
        out_shape=jax.ShapeDtypeStruct((GRAPHS, OUT_DIM), jnp.float32),
    )(pooled_parts, w1, b1[None], w2, b2[None])


def _gcn(x, src, dst, ew, W, b, n):
    loop = jnp.arange(n, dtype=src.dtype)
    s = jnp.concatenate([src, loop])
    d = jnp.concatenate([dst, loop])
    w = jnp.concatenate([ew, jnp.ones((n,), dtype=ew.dtype)])
    deg = jax.ops.segment_sum(w, d, num_segments=n)
    dinv = jnp.where(deg > 0, 1.0 / jnp.sqrt(deg), 0.0)
    norm = dinv[s] * w * dinv[d]
    h = x @ W
    out = jax.ops.segment_sum(h[s] * norm[:, None], d, num_segments=n)
    return out + b


def _gat(x, src, dst, W, a_s, a_d, b, n):
    h = (x @ W).reshape(n, HEADS, HID)
    al_s = jnp.sum(h * a_s[None], axis=-1)
    al_d = jnp.sum(h * a_d[None], axis=-1)
    e = jax.nn.leaky_relu(al_s[src] + al_d[dst], negative_slope=0.2)
    ex = jnp.exp(e)
    ex_self = jnp.exp(jax.nn.leaky_relu(al_s + al_d, negative_slope=0.2))
    z = jax.ops.segment_sum(ex, dst, num_segments=n) + ex_self
    zinv = 1.0 / z
    alpha = ex * zinv[dst]
    num = jax.ops.segment_sum(h[src] * alpha[:, :, None], dst, num_segments=n)
    num = num + h * (ex_self * zinv)[:, :, None]
    return num.mean(axis=1) + b


def kernel(x, edge_index, edge_weight, batch, W_gcn, b_gcn, g1_W, g1_as, g1_ad, g1_b, g2_W, g2_as, g2_ad, g2_b, fc1_W, fc1_b, fc2_W, fc2_b, pro_bias, gfc1_W, gfc1_b, gfc2_W, gfc2_b):
    src = edge_index[0]
    dst = edge_index[1]
    n = x.shape[0]
    xc = _gcn(x, src, dst, edge_weight, W_gcn, b_gcn, n)
    xcur = jax.nn.relu(xc)
    xc = _gat(xcur, src, dst, g1_W, g1_as, g1_ad, g1_b, n)
    xc = jax.nn.relu(xc)
    z = jax.nn.sigmoid(xc @ fc1_W + fc1_b + xcur @ fc2_W + fc2_b + pro_bias)
    xcur = z * xc + (1.0 - z) * xcur
    xc = _gat(xcur, src, dst, g2_W, g2_as, g2_ad, g2_b, n)
    z = jax.nn.sigmoid(xc @ fc1_W + fc1_b + xcur @ fc2_W + fc2_b + pro_bias)
    xcur = z * xc + (1.0 - z) * xcur
    pooled = jax.ops.segment_max(xcur, batch, num_segments=GRAPHS)
    return _mlp_head(pooled[None], gfc1_W, gfc1_b, gfc2_W, gfc2_b)


# speedup vs baseline: 24.8719x; 22.2097x over previous
import functools

import jax
import jax.numpy as jnp
from jax import lax
from jax.experimental import pallas as pl
from jax.experimental.pallas import tpu as pltpu
from jax.experimental.pallas import tpu_sc as plsc

N = 50000
E = 800000
F_IN = 16
HID = 64
HEADS = 4
GRAPHS = 256
OUT_DIM = 128

NC = 2
NT = 16
L = 16
CH = 128
NCHUNK = E // CH
PER = 3128
LAST = N - PER * (NT - 1)
BN = 2000
NEG = -3.0e38


def _acc_zero(zsrc, acc, tid):
    @pl.when(tid < NT - 1)
    def _():
        o = pl.multiple_of(tid * PER, 8)
        pltpu.sync_copy(zsrc.at[pl.ds(o, PER)], acc.at[pl.ds(o, PER)])

    @pl.when(tid == NT - 1)
    def _():
        pltpu.sync_copy(zsrc.at[pl.ds(PER * (NT - 1), LAST)],
                        acc.at[pl.ds(PER * (NT - 1), LAST)])


def _acc_out(acc, out, tid, cid):
    @pl.when(tid < NT - 1)
    def _():
        o = pl.multiple_of(tid * PER, 8)
        oo = pl.multiple_of(cid * N + tid * PER, 8)
        pltpu.sync_copy(acc.at[pl.ds(o, PER)], out.at[pl.ds(oo, PER)])

    @pl.when(tid == NT - 1)
    def _():
        oo = pl.multiple_of(cid * N + PER * (NT - 1), 8)
        pltpu.sync_copy(acc.at[pl.ds(PER * (NT - 1), LAST)], out.at[pl.ds(oo, LAST)])


def _vsc_mesh():
    return plsc.VectorSubcoreMesh(
        core_axis_name="c", subcore_axis_name="s", num_cores=NC, num_subcores=NT)


_SC_PARAMS = pltpu.CompilerParams(use_tc_tiling_on_sc=False)

def _mm(a, b):
    return jax.lax.dot_general(a, b, (((a.ndim - 1,), (0,)), ((), ())),
                               precision=jax.lax.Precision.HIGHEST)


def _deg_body(dsth, w16h, z16, out, dacc, dsti, wrow):
    cid = lax.axis_index("c")
    tid = lax.axis_index("s")
    wid = tid * NC + cid
    _acc_zero(z16, dacc, tid)
    plsc.subcore_barrier()

    nloop = NCHUNK // (NC * NT) + 1

    def chunk(j, carry):
        k = j * (NC * NT) + wid

        @pl.when(k < NCHUNK)
        def _():
            base = k * CH
            pltpu.sync_copy(dsth.at[pl.ds(base, CH)], dsti)
            pltpu.sync_copy(w16h.at[pl.ds(base, CH)], wrow)
            pltpu.sync_copy(wrow, dacc.at[dsti], add=True)
        return carry

    lax.fori_loop(0, nloop, chunk, 0)
    plsc.subcore_barrier()
    _acc_out(dacc, out, tid, cid)


def _deg_pass(dst, w16, z16):
    f = pl.kernel(
        _deg_body,
        out_type=jax.ShapeDtypeStruct((NC * N, L), jnp.float32),
        mesh=_vsc_mesh(),
        compiler_params=_SC_PARAMS,
        scratch_types=[
            pltpu.VMEM_SHARED((N, L), jnp.float32),
            pltpu.VMEM((CH,), jnp.int32),
            pltpu.VMEM((CH, L), jnp.float32),
        ],
    )
    return f(dst, w16, z16)


def _gcn_body(hp, srch, dsth, wh, z32, out, acc, srci, dsti, wv, rows):
    cid = lax.axis_index("c")
    tid = lax.axis_index("s")
    iota = lax.iota(jnp.int32, L)
    _acc_zero(z32, acc, tid)
    plsc.subcore_barrier()

    nloop = NCHUNK // NT + 1
    off = cid * N

    def chunk(j, carry):
        k = j * NT + tid

        @pl.when(k < NCHUNK)
        def _():
            base = k * CH
            pltpu.sync_copy(srch.at[pl.ds(base, CH)], srci)
            pltpu.sync_copy(dsth.at[pl.ds(base, CH)], dsti)
            pltpu.sync_copy(wh.at[pl.ds(base, CH)], wv)
            for i in range(CH // L):
                srci[pl.ds(i * L, L)] = srci[pl.ds(i * L, L)] + off
            pltpu.sync_copy(hp.at[srci], rows)

            def group(g, c2):
                wvec = wv[pl.ds(g * L, L)]
                for e2 in range(L):
                    wb = lax.broadcast_in_dim(wvec[e2], (L,), ())
                    e = g * L + e2
                    for k2 in range(2):
                        v = rows[e, pl.ds(k2 * L, L)]
                        rows[e, pl.ds(k2 * L, L)] = v * wb
                return c2

            lax.fori_loop(0, CH // L, group, 0)
            pltpu.sync_copy(rows, acc.at[dsti], add=True)
        return carry

    lax.fori_loop(0, nloop, chunk, 0)
    plsc.subcore_barrier()
    _acc_out(acc, out, tid, cid)


def _gcn_pass(hp, src, dst, w, z32):
    f = pl.kernel(
        _gcn_body,
        out_type=jax.ShapeDtypeStruct((NC * N, 32), jnp.float32),
        mesh=_vsc_mesh(),
        compiler_params=_SC_PARAMS,
        scratch_types=[
            pltpu.VMEM_SHARED((N, 32), jnp.float32),
            pltpu.VMEM((CH,), jnp.int32),
            pltpu.VMEM((CH,), jnp.int32),
            pltpu.VMEM((CH,), jnp.float32),
            pltpu.VMEM((CH, 32), jnp.float32),
        ],
    )
    return f(hp, src, dst, w, z32)


def _gat_logit_body(alsh, aldh, srch, dsth, z32, out, exb, zacc, srci, dsti,
                    asrc, adst, exch):
    cid = lax.axis_index("c")
    tid = lax.axis_index("s")
    wid = tid * NC + cid
    iota = lax.iota(jnp.int32, L)
    lanelt4 = iota < 4
    _acc_zero(z32, zacc, tid)
    pltpu.sync_copy(z32.at[pl.ds(0, CH)], exch)
    plsc.subcore_barrier()

    nloop = NCHUNK // (NC * NT) + 1

    def chunk(j, carry):
        k = j * (NC * NT) + wid

        @pl.when(k < NCHUNK)
        def _():
            base = k * CH
            pltpu.sync_copy(srch.at[pl.ds(base, CH)], srci)
            pltpu.sync_copy(dsth.at[pl.ds(base, CH)], dsti)
            pltpu.sync_copy(alsh.at[srci], asrc)
            pltpu.sync_copy(aldh.at[dsti], adst)

            def edge16(g, c2):
                for e2 in range(L):
                    e = g * L + e2
                    ev = asrc[e, pl.ds(0, L)] + adst[e, pl.ds(0, L)]
                    ev = jnp.where(ev > 0, ev, 0.2 * ev)
                    ex = jnp.where(lanelt4, jnp.exp(ev), 0.0)
                    exch[e, pl.ds(0, L)] = ex
                return c2

            lax.fori_loop(0, CH // L, edge16, 0)
            pltpu.sync_copy(exch, zacc.at[dsti], add=True)
            pltpu.sync_copy(exch.at[pl.ds(0, CH), pl.ds(0, L)],
                            exb.at[pl.ds(base, CH)])
        return carry

    lax.fori_loop(0, nloop, chunk, 0)
    plsc.subcore_barrier()
    _acc_out(zacc, out, tid, cid)


def _gat_logit_pass(als16, ald16, src, dst, z32):
    f = pl.kernel(
        _gat_logit_body,
        out_type=[
            jax.ShapeDtypeStruct((NC * N, 32), jnp.float32),
            jax.ShapeDtypeStruct((E, L), jnp.float32),
        ],
        mesh=_vsc_mesh(),
        compiler_params=_SC_PARAMS,
        scratch_types=[
            pltpu.VMEM_SHARED((N, 32), jnp.float32),
            pltpu.VMEM((CH,), jnp.int32),
            pltpu.VMEM((CH,), jnp.int32),
            pltpu.VMEM((CH, L), jnp.float32),
            pltpu.VMEM((CH, L), jnp.float32),
            pltpu.VMEM((CH, 32), jnp.float32),
        ],
    )
    return f(als16, ald16, src, dst, z32)


def _gat_out_body(hsc, srch, dsth, exbh, zinvh, z32, out, acc, srci, dsti,
                  exch, zrows, hrows, orows):
    cid = lax.axis_index("c")
    tid = lax.axis_index("s")
    _acc_zero(z32, acc, tid)
    plsc.subcore_barrier()

    nloop = NCHUNK // NT + 1
    off = cid * N

    def chunk(j, carry):
        k = j * NT + tid

        @pl.when(k < NCHUNK)
        def _():
            base = k * CH
            pltpu.sync_copy(srch.at[pl.ds(base, CH)], srci)
            pltpu.sync_copy(dsth.at[pl.ds(base, CH)], dsti)
            pltpu.sync_copy(exbh.at[pl.ds(base, CH)], exch)
            for i in range(CH // L):
                srci[pl.ds(i * L, L)] = srci[pl.ds(i * L, L)] + off
            pltpu.sync_copy(hsc.at[srci], hrows)
            pltpu.sync_copy(zinvh.at[dsti], zrows)

            def edge16(g, c2):
                for e2 in range(L):
                    e = g * L + e2
                    av = exch[e, pl.ds(0, L)] * zrows[e, pl.ds(0, L)]
                    o0 = lax.broadcast_in_dim(av[0], (L,), ()) * hrows[e, pl.ds(0, L)]
                    o1 = lax.broadcast_in_dim(av[0], (L,), ()) * hrows[e, pl.ds(L, L)]
                    for h in range(1, 4):
                        ab = lax.broadcast_in_dim(av[h], (L,), ())
                        o0 = o0 + ab * hrows[e, pl.ds(h * 32, L)]
                        o1 = o1 + ab * hrows[e, pl.ds(h * 32 + L, L)]
                    orows[e, pl.ds(0, L)] = o0
                    orows[e, pl.ds(L, L)] = o1
                return c2

            lax.fori_loop(0, CH // L, edge16, 0)
            pltpu.sync_copy(orows, acc.at[dsti], add=True)
        return carry

    lax.fori_loop(0, nloop, chunk, 0)
    plsc.subcore_barrier()
    _acc_out(acc, out, tid, cid)


def _gat_out_pass(hsc, src, dst, exb, zinv16, z32):
    f = pl.kernel(
        _gat_out_body,
        out_type=jax.ShapeDtypeStruct((NC * N, 32), jnp.float32),
        mesh=_vsc_mesh(),
        compiler_params=_SC_PARAMS,
        scratch_types=[
            pltpu.VMEM_SHARED((N, 32), jnp.float32),
            pltpu.VMEM((CH,), jnp.int32),
            pltpu.VMEM((CH,), jnp.int32),
            pltpu.VMEM((CH, L), jnp.float32),
            pltpu.VMEM((CH, L), jnp.float32),
            pltpu.VMEM((CH, 128), jnp.float32),
            pltpu.VMEM((CH, 32), jnp.float32),
        ],
    )
    return f(hsc, src, dst, exb, zinv16, z32)


RCH = 400


def _pool_body(xh, bh, out, acc, xrows, bidx):
    cid = lax.axis_index("c")
    tid = lax.axis_index("s")
    wid = tid * NC + cid
    negv = jnp.full((L,), NEG, jnp.float32)

    def initrow(r, c):
        for k in range(HID // L):
            acc[r, pl.ds(k * L, L)] = negv
        return c

    lax.fori_loop(0, GRAPHS, initrow, 0)

    nloop = (N // RCH) // (NC * NT) + 1

    def chunk(j, carry):
        k = j * (NC * NT) + wid

        @pl.when(k < N // RCH)
        def _():
            base = k * RCH
            pltpu.sync_copy(xh.at[pl.ds(base, RCH)], xrows)
            pltpu.sync_copy(bh.at[pl.ds(base, RCH)], bidx)

            def row16(g, c2):
                bvec = bidx[pl.ds(g * L, L)]
                for r2 in range(L):
                    gid = bvec[r2]
                    r = g * L + r2
                    for k2 in range(HID // L):
                        cur = acc[gid, pl.ds(k2 * L, L)]
                        xv = xrows[r, pl.ds(k2 * L, L)]
                        acc[gid, pl.ds(k2 * L, L)] = jnp.maximum(cur, xv)
                return c2

            lax.fori_loop(0, RCH // L, row16, 0)
        return carry

    lax.fori_loop(0, nloop, chunk, 0)
    pltpu.sync_copy(acc, out.at[pl.ds(wid * GRAPHS, GRAPHS)])


def _pool_pass(xcur, batch):
    f = pl.kernel(
        _pool_body,
        out_type=jax.ShapeDtypeStruct((NC * NT * GRAPHS, HID), jnp.float32),
        mesh=_vsc_mesh(),
        compiler_params=_SC_PARAMS,
        scratch_types=[
            pltpu.VMEM((GRAPHS, HID), jnp.float32),
            pltpu.VMEM((RCH, HID), jnp.float32),
            pltpu.VMEM((RCH,), jnp.int32),
        ],
    )
    return f(xcur, batch)


def _tc2_body(x_ref, w_ref, degp_ref, hp_ref, dinv_ref):
    h1 = _mm(x_ref[...], w_ref[...])
    deg = 1.0 + degp_ref[0, :, 0] + degp_ref[1, :, 0]
    dinv = jnp.where(deg > 0, lax.rsqrt(deg), 0.0)
    hp = h1 * dinv[:, None]
    hp_ref[0] = hp[:, :32]
    hp_ref[1] = hp[:, 32:]
    dinv_ref[...] = jnp.broadcast_to(dinv[:, None], (BN, 8))


def _tc2(x, W_gcn, degp):
    return pl.pallas_call(
        _tc2_body,
        grid=(N // BN,),
        in_specs=[
            pl.BlockSpec((BN, F_IN), lambda i: (i, 0)),
            pl.BlockSpec((F_IN, HID), lambda i: (0, 0)),
            pl.BlockSpec((2, BN, L), lambda i: (0, i, 0)),
        ],
        out_specs=[
            pl.BlockSpec((2, BN, 32), lambda i: (0, i, 0)),
            pl.BlockSpec((BN, 8), lambda i: (i, 0)),
        ],
        out_shape=[
            jax.ShapeDtypeStruct((2, N, 32), jnp.float32),
            jax.ShapeDtypeStruct((N, 8), jnp.float32),
        ],
    )(x, W_gcn, degp)


def _gat_prep(H, asc_ref, adc_ref, g128, hsc_ref, als_ref, ald_ref):
    als = jnp.zeros((BN, 4), jnp.float32)
    ald = jnp.zeros((BN, 4), jnp.float32)
    for c in range(2):
        hc = jnp.concatenate(
            [H[:, h * 64 + c * 32:h * 64 + (c + 1) * 32] for h in range(4)],
            axis=1)
        hsc_ref[c] = hc
        als = als + _mm(hc * asc_ref[c][None, :], g128)
        ald = ald + _mm(hc * adc_ref[c][None, :], g128)
    pad = jnp.zeros((BN, 12), jnp.float32)
    als_ref[...] = jnp.concatenate([als, pad], axis=1)
    ald_ref[...] = jnp.concatenate([ald, pad], axis=1)


def _tc3_body(gacc_ref, hp_ref, dinv_ref, b_ref, gw_ref, asc_ref, adc_ref,
              g128_ref, xcur_ref, hsc_ref, als_ref, ald_ref):
    acc = jnp.concatenate([gacc_ref[0], gacc_ref[1]], axis=1)
    hpv = jnp.concatenate([hp_ref[0], hp_ref[1]], axis=1)
    dinv = dinv_ref[:, 0]
    xcur = jax.nn.relu((acc + hpv) * dinv[:, None] + b_ref[...])
    xcur_ref[...] = xcur
    H = _mm(xcur, gw_ref[...])
    _gat_prep(H, asc_ref, adc_ref, g128_ref[...], hsc_ref, als_ref, ald_ref)


def _tc3(gacc, hp, dinv8, b_gcn, gw, ascf, adcf, g128):
    return pl.pallas_call(
        _tc3_body,
        grid=(N // BN,),
        in_specs=[
            pl.BlockSpec((2, BN, 32), lambda i: (0, i, 0)),
            pl.BlockSpec((2, BN, 32), lambda i: (0, i, 0)),
            pl.BlockSpec((BN, 8), lambda i: (i, 0)),
            pl.BlockSpec((1, HID), lambda i: (0, 0)),
            pl.BlockSpec((HID, HEADS * HID), lambda i: (0, 0)),
            pl.BlockSpec((2, 128), lambda i: (0, 0)),
            pl.BlockSpec((2, 128), lambda i: (0, 0)),
            pl.BlockSpec((128, 4), lambda i: (0, 0)),
        ],
        out_specs=[
            pl.BlockSpec((BN, HID), lambda i: (i, 0)),
            pl.BlockSpec((2, BN, 128), lambda i: (0, i, 0)),
            pl.BlockSpec((BN, 16), lambda i: (i, 0)),
            pl.BlockSpec((BN, 16), lambda i: (i, 0)),
        ],
        out_shape=[
            jax.ShapeDtypeStruct((N, HID), jnp.float32),
            jax.ShapeDtypeStruct((2, N, 128), jnp.float32),
            jax.ShapeDtypeStruct((N, 16), jnp.float32),
            jax.ShapeDtypeStruct((N, 16), jnp.float32),
        ],
    )(gacc, hp, dinv8, b_gcn, gw, ascf, adcf, g128)


def _tc4_body(zp_ref, als_ref, ald_ref, zinv_ref, selfa_ref):
    al = als_ref[:, :4] + ald_ref[:, :4]
    ex_self = jnp.exp(jnp.where(al > 0, al, 0.2 * al))
    z = zp_ref[0, :, :4] + zp_ref[1, :, :4] + ex_self
    zinv = 1.0 / z
    pad = jnp.zeros((BN, 12), jnp.float32)
    zinv_ref[...] = jnp.concatenate([zinv, pad], axis=1)
    selfa_ref[...] = jnp.concatenate([ex_self * zinv, pad], axis=1)


def _tc4(zp, als16, ald16):
    return pl.pallas_call(
        _tc4_body,
        grid=(N // BN,),
        in_specs=[
            pl.BlockSpec((2, BN, 32), lambda i: (0, i, 0)),
            pl.BlockSpec((BN, 16), lambda i: (i, 0)),
            pl.BlockSpec((BN, 16), lambda i: (i, 0)),
        ],
        out_specs=[
            pl.BlockSpec((BN, 16), lambda i: (i, 0)),
            pl.BlockSpec((BN, 16), lambda i: (i, 0)),
        ],
        out_shape=[
            jax.ShapeDtypeStruct((N, 16), jnp.float32),
            jax.ShapeDtypeStruct((N, 16), jnp.float32),
        ],
    )(zp, als16, ald16)


def _tc5_layer1(accp, hsc, selfa16, xcur, g_b, g4, r128,
                fc1_W, fc1_b, fc2_W, fc2_b, pro_bias,
                gw2, asc2, adc2, g128):
    def body(accp_ref, hsc_ref, selfa_ref, xcur_ref, gb_ref, g4_ref, r128_ref,
             fc1w_ref, fc1b_ref, fc2w_ref, fc2b_ref, pro_ref,
             gw2_ref, asc2_ref, adc2_ref, g128_ref,
             xnew_ref, hsc2_ref, als2_ref, ald2_ref):
        selfexp = _mm(selfa_ref[...], g4_ref[...])
        halves = []
        for c in range(2):
            self_c = _mm(hsc_ref[c] * selfexp, r128_ref[...])
            halves.append(0.25 * (accp_ref[c] + self_c))
        xc = jax.nn.relu(jnp.concatenate(halves, axis=1) + gb_ref[...])
        xcur = xcur_ref[...]
        zg = jax.nn.sigmoid(_mm(xc, fc1w_ref[...]) + fc1b_ref[...]
                            + _mm(xcur, fc2w_ref[...]) + fc2b_ref[...] + pro_ref[...])
        xnew = zg * xc + (1.0 - zg) * xcur
        xnew_ref[...] = xnew
        H2 = _mm(xnew, gw2_ref[...])
        _gat_prep(H2, asc2_ref, adc2_ref, g128_ref[...], hsc2_ref, als2_ref, ald2_ref)

    return pl.pallas_call(
        body,
        grid=(N // BN,),
        in_specs=[
            pl.BlockSpec((2, BN, 32), lambda i: (0, i, 0)),
            pl.BlockSpec((2, BN, 128), lambda i: (0, i, 0)),
            pl.BlockSpec((BN, 16), lambda i: (i, 0)),
            pl.BlockSpec((BN, HID), lambda i: (i, 0)),
            pl.BlockSpec((1, HID), lambda i: (0, 0)),
            pl.BlockSpec((16, 128), lambda i: (0, 0)),
            pl.BlockSpec((128, 32), lambda i: (0, 0)),
            pl.BlockSpec((HID, HID), lambda i: (0, 0)),
            pl.BlockSpec((1, HID), lambda i: (0, 0)),
            pl.BlockSpec((HID, HID), lambda i: (0, 0)),
            pl.BlockSpec((1, HID), lambda i: (0, 0)),
            pl.BlockSpec((1, HID), lambda i: (0, 0)),
            pl.BlockSpec((HID, HEADS * HID), lambda i: (0, 0)),
            pl.BlockSpec((2, 128), lambda i: (0, 0)),
            pl.BlockSpec((2, 128), lambda i: (0, 0)),
            pl.BlockSpec((128, 4), lambda i: (0, 0)),
        ],
        out_specs=[
            pl.BlockSpec((BN, HID), lambda i: (i, 0)),
            pl.BlockSpec((2, BN, 128), lambda i: (0, i, 0)),
            pl.BlockSpec((BN, 16), lambda i: (i, 0)),
            pl.BlockSpec((BN, 16), lambda i: (i, 0)),
        ],
        out_shape=[
            jax.ShapeDtypeStruct((N, HID), jnp.float32),
            jax.ShapeDtypeStruct((2, N, 128), jnp.float32),
            jax.ShapeDtypeStruct((N, 16), jnp.float32),
            jax.ShapeDtypeStruct((N, 16), jnp.float32),
        ],
    )(accp, hsc, selfa16, xcur, g_b, g4, r128, fc1_W, fc1_b, fc2_W, fc2_b,
      pro_bias, gw2, asc2, adc2, g128)


def _tc5_layer2(accp, hsc, selfa16, xcur, g_b, g4, r128,
                fc1_W, fc1_b, fc2_W, fc2_b, pro_bias):
    def body(accp_ref, hsc_ref, selfa_ref, xcur_ref, gb_ref, g4_ref, r128_ref,
             fc1w_ref, fc1b_ref, fc2w_ref, fc2b_ref, pro_ref, xnew_ref):
        selfexp = _mm(selfa_ref[...], g4_ref[...])
        halves = []
        for c in range(2):
            self_c = _mm(hsc_ref[c] * selfexp, r128_ref[...])
            halves.append(0.25 * (accp_ref[c] + self_c))
        xc = jnp.concatenate(halves, axis=1) + gb_ref[...]
        xcur = xcur_ref[...]
        zg = jax.nn.sigmoid(_mm(xc, fc1w_ref[...]) + fc1b_ref[...]
                            + _mm(xcur, fc2w_ref[...]) + fc2b_ref[...] + pro_ref[...])
        xnew_ref[...] = zg * xc + (1.0 - zg) * xcur

    return pl.pallas_call(
        body,
        grid=(N // BN,),
        in_specs=[
            pl.BlockSpec((2, BN, 32), lambda i: (0, i, 0)),
            pl.BlockSpec((2, BN, 128), lambda i: (0, i, 0)),
            pl.BlockSpec((BN, 16), lambda i: (i, 0)),
            pl.BlockSpec((BN, HID), lambda i: (i, 0)),
            pl.BlockSpec((1, HID), lambda i: (0, 0)),
            pl.BlockSpec((16, 128), lambda i: (0, 0)),
            pl.BlockSpec((128, 32), lambda i: (0, 0)),
            pl.BlockSpec((HID, HID), lambda i: (0, 0)),
            pl.BlockSpec((1, HID), lambda i: (0, 0)),
            pl.BlockSpec((HID, HID), lambda i: (0, 0)),
            pl.BlockSpec((1, HID), lambda i: (0, 0)),
            pl.BlockSpec((1, HID), lambda i: (0, 0)),
        ],
        out_specs=[pl.BlockSpec((BN, HID), lambda i: (i, 0))],
        out_shape=[jax.ShapeDtypeStruct((N, HID), jnp.float32)],
    )(accp, hsc, selfa16, xcur, g_b, g4, r128, fc1_W, fc1_b, fc2_W, fc2_b,
      pro_bias)


def _mlp_head_body(pooled_parts_ref, w1_ref, b1_ref, w2_ref, b2_ref, out_ref):
    pooled = jnp.max(pooled_parts_ref[...], axis=0)
    pooled = jnp.where(pooled > -1.0e38, pooled, 0.0)
    h = jax.nn.relu(_mm(pooled, w1_ref[...]) + b1_ref[...])
    out_ref[...] = _mm(h, w2_ref[...]) + b2_ref[...]


def _mlp_head(pooled_parts, w1, b1, w2, b2):
    return pl.pallas_call(
        _mlp_head_body,
        out_shape=jax.ShapeDtypeStruct((GRAPHS, OUT_DIM), jnp.float32),
    )(pooled_parts, w1, b1[None], w2, b2[None])


def _prep_a(a):
    return a.reshape(4, 2, 32).transpose(1, 0, 2).reshape(2, 128)


def kernel(x, edge_index, edge_weight, batch, W_gcn, b_gcn, g1_W, g1_as, g1_ad, g1_b, g2_W, g2_as, g2_ad, g2_b, fc1_W, fc1_b, fc2_W, fc2_b, pro_bias, gfc1_W, gfc1_b, gfc2_W, gfc2_b):
    src = edge_index[0]
    dst = edge_index[1]
    z16 = jnp.zeros((N, L), jnp.float32)
    z32 = jnp.zeros((N, 32), jnp.float32)
    g128 = (jnp.arange(128)[:, None] // 32 == jnp.arange(4)[None, :]).astype(jnp.float32)
    g4 = (jnp.arange(16)[:, None] == jnp.arange(128)[None, :] // 32).astype(jnp.float32)
    r128 = (jnp.arange(128)[:, None] % 32 == jnp.arange(32)[None, :]).astype(jnp.float32)

    w16 = jnp.broadcast_to(edge_weight[:, None], (E, L))
    degp = _deg_pass(dst, w16, z16).reshape(2, N, L)
    hp2, dinv8 = _tc2(x, W_gcn, degp)
    gacc = _gcn_pass(hp2.reshape(2 * N, 32), src, dst, edge_weight, z32)
    xcur, Hsc, als16, ald16 = _tc3(gacc.reshape(2, N, 32), hp2, dinv8,
                                   b_gcn[None], g1_W, _prep_a(g1_as),
                                   _prep_a(g1_ad), g128)

    zp, exb = _gat_logit_pass(als16, ald16, src, dst, z32)
    zinv16, selfa16 = _tc4(zp.reshape(2, N, 32), als16, ald16)
    acc1 = _gat_out_pass(Hsc.reshape(2 * N, 128), src, dst, exb, zinv16, z32)
    xcur, Hsc2, als26, ald26 = _tc5_layer1(
        acc1.reshape(2, N, 32), Hsc, selfa16, xcur, g1_b[None], g4, r128,
        fc1_W, fc1_b[None], fc2_W, fc2_b[None], pro_bias[None],
        g2_W, _prep_a(g2_as), _prep_a(g2_ad), g128)

    zp2, exb2 = _gat_logit_pass(als26, ald26, src, dst, z32)
    zinv26, selfa26 = _tc4(zp2.reshape(2, N, 32), als26, ald26)
    acc2 = _gat_out_pass(Hsc2.reshape(2 * N, 128), src, dst, exb2, zinv26, z32)
    (xcur,) = _tc5_layer2(
        acc2.reshape(2, N, 32), Hsc2, selfa26, xcur, g2_b[None], g4, r128,
        fc1_W, fc1_b[None], fc2_W, fc2_b[None], pro_bias[None])

    parts = _pool_pass(xcur, batch).reshape(NC * NT, GRAPHS, HID)
    return _mlp_head(parts, gfc1_W, gfc1_b, gfc2_W, gfc2_b)


# speedup vs baseline: 27.3949x; 1.1014x over previous
import functools

import jax
import jax.numpy as jnp
from jax import lax
from jax.experimental import pallas as pl
from jax.experimental.pallas import tpu as pltpu
from jax.experimental.pallas import tpu_sc as plsc

N = 50000
E = 800000
F_IN = 16
HID = 64
HEADS = 4
GRAPHS = 256
OUT_DIM = 128

NC = 2
NT = 16
L = 16
CH = 128
NCHUNK = E // CH
EP = 802816
EPC = EP // CH
CPT = EPC // NT
CPW = EPC // (NC * NT)
NA = N + 8
CHO = 64
CPTO = (EP // CHO) // NT
PER = 3128
LAST = N - PER * (NT - 1)
BN = 2000
NEG = -3.0e38


def _acc_zero(zsrc, acc, tid):
    @pl.when(tid < NT - 1)
    def _():
        o = pl.multiple_of(tid * PER, 8)
        pltpu.sync_copy(zsrc.at[pl.ds(o, PER)], acc.at[pl.ds(o, PER)])

    @pl.when(tid == NT - 1)
    def _():
        pltpu.sync_copy(zsrc.at[pl.ds(PER * (NT - 1), LAST)],
                        acc.at[pl.ds(PER * (NT - 1), LAST)])


def _acc_out(acc, out, tid, cid):
    @pl.when(tid < NT - 1)
    def _():
        o = pl.multiple_of(tid * PER, 8)
        oo = pl.multiple_of(cid * N + tid * PER, 8)
        pltpu.sync_copy(acc.at[pl.ds(o, PER)], out.at[pl.ds(oo, PER)])

    @pl.when(tid == NT - 1)
    def _():
        oo = pl.multiple_of(cid * N + PER * (NT - 1), 8)
        pltpu.sync_copy(acc.at[pl.ds(PER * (NT - 1), LAST)], out.at[pl.ds(oo, LAST)])


def _vsc_mesh():
    return plsc.VectorSubcoreMesh(
        core_axis_name="c", subcore_axis_name="s", num_cores=NC, num_subcores=NT)


_SC_PARAMS = pltpu.CompilerParams(use_tc_tiling_on_sc=False)

def _mm(a, b):
    return jax.lax.dot_general(a, b, (((a.ndim - 1,), (0,)), ((), ())),
                               precision=jax.lax.Precision.HIGHEST)


def _deg_body(dsth, w16h, z16, out, dacc, dsti, wrow):
    cid = lax.axis_index("c")
    tid = lax.axis_index("s")
    wid = tid * NC + cid
    _acc_zero(z16, dacc, tid)
    plsc.subcore_barrier()

    def chunk(j, carry):
        base = (j * (NC * NT) + wid) * CH
        pltpu.sync_copy(dsth.at[pl.ds(base, CH)], dsti)
        pltpu.sync_copy(w16h.at[pl.ds(base, CH)], wrow)
        pltpu.sync_copy(wrow, dacc.at[dsti], add=True)
        return carry

    lax.fori_loop(0, CPW, chunk, 0)
    plsc.subcore_barrier()
    _acc_out(dacc, out, tid, cid)


def _deg_pass(dst, w16, z16):
    f = pl.kernel(
        _deg_body,
        out_type=jax.ShapeDtypeStruct((NC * N, L), jnp.float32),
        mesh=_vsc_mesh(),
        compiler_params=_SC_PARAMS,
        scratch_types=[
            pltpu.VMEM_SHARED((NA, L), jnp.float32),
            pltpu.VMEM((CH,), jnp.int32),
            pltpu.VMEM((CH, L), jnp.float32),
        ],
    )
    return f(dst, w16, z16)


def _gcn_body(hp, srch, dsth, wh, z32, out, acc,
              srci0, srci1, dsti0, dsti1, wv0, wv1, rows0, rows1, sg0, sg1):
    cid = lax.axis_index("c")
    tid = lax.axis_index("s")
    _acc_zero(z32, acc, tid)
    plsc.subcore_barrier()
    off = cid * N
    base0 = tid * CPT
    srcis = (srci0, srci1)
    dstis = (dsti0, dsti1)
    wvs = (wv0, wv1)
    rowss = (rows0, rows1)
    sgs = (sg0, sg1)

    def load_idx(b, c):
        base = c * CH
        pltpu.sync_copy(srch.at[pl.ds(base, CH)], srcis[b])
        pltpu.sync_copy(dsth.at[pl.ds(base, CH)], dstis[b])
        pltpu.sync_copy(wh.at[pl.ds(base, CH)], wvs[b])
        for i in range(CH // L):
            srcis[b][pl.ds(i * L, L)] = srcis[b][pl.ds(i * L, L)] + off

    def issue(b):
        pltpu.async_copy(hp.at[srcis[b]], rowss[b], sgs[b])

    def wait(b):
        pltpu.make_async_copy(hp.at[srcis[b]], rowss[b], sgs[b]).wait()

    def compute(b):
        rows, wv, dsti = rowss[b], wvs[b], dstis[b]

        def group(g, c2):
            wvec = wv[pl.ds(g * L, L)]
            for e2 in range(L):
                wb = lax.broadcast_in_dim(wvec[e2], (L,), ())
                e = g * L + e2
                for k2 in range(2):
                    v = rows[e, pl.ds(k2 * L, L)]
                    rows[e, pl.ds(k2 * L, L)] = v * wb
            return c2

        lax.fori_loop(0, CH // L, group, 0)
        pltpu.sync_copy(rows, acc.at[dsti], add=True)

    load_idx(0, base0)
    issue(0)

    def pair(j2, carry):
        c = base0 + 2 * j2
        load_idx(1, c + 1)
        wait(0)
        issue(1)
        compute(0)
        @pl.when(j2 < CPT // 2 - 1)
        def _():
            load_idx(0, c + 2)
        wait(1)

        @pl.when(j2 < CPT // 2 - 1)
        def _():
            issue(0)
        compute(1)
        return carry

    lax.fori_loop(0, CPT // 2, pair, 0)
    plsc.subcore_barrier()
    _acc_out(acc, out, tid, cid)


def _gcn_pass(hp, src, dst, w, z32):
    f = pl.kernel(
        _gcn_body,
        out_type=jax.ShapeDtypeStruct((NC * N, 32), jnp.float32),
        mesh=_vsc_mesh(),
        compiler_params=_SC_PARAMS,
        scratch_types=[
            pltpu.VMEM_SHARED((NA, 32), jnp.float32),
            pltpu.VMEM((CH,), jnp.int32),
            pltpu.VMEM((CH,), jnp.int32),
            pltpu.VMEM((CH,), jnp.int32),
            pltpu.VMEM((CH,), jnp.int32),
            pltpu.VMEM((CH,), jnp.float32),
            pltpu.VMEM((CH,), jnp.float32),
            pltpu.VMEM((CH, 32), jnp.float32),
            pltpu.VMEM((CH, 32), jnp.float32),
            pltpu.SemaphoreType.DMA,
            pltpu.SemaphoreType.DMA,
        ],
    )
    return f(hp, src, dst, w, z32)


def _gat_logit_body(alsh, aldh, srch, dsth, z32, out, exb, zacc, srci, dsti,
                    asrc, adst, exch):
    cid = lax.axis_index("c")
    tid = lax.axis_index("s")
    wid = tid * NC + cid
    iota = lax.iota(jnp.int32, L)
    lanelt4 = iota < 4
    _acc_zero(z32, zacc, tid)
    pltpu.sync_copy(z32.at[pl.ds(0, CH)], exch)
    plsc.subcore_barrier()

    def chunk(j, carry):
        base = (j * (NC * NT) + wid) * CH
        pltpu.sync_copy(srch.at[pl.ds(base, CH)], srci)
        pltpu.sync_copy(dsth.at[pl.ds(base, CH)], dsti)
        pltpu.sync_copy(alsh.at[srci], asrc)
        pltpu.sync_copy(aldh.at[dsti], adst)

        def edge16(g, c2):
            for e2 in range(L):
                e = g * L + e2
                ev = asrc[e, pl.ds(0, L)] + adst[e, pl.ds(0, L)]
                ev = jnp.where(ev > 0, ev, 0.2 * ev)
                ex = jnp.where(lanelt4, jnp.exp(ev), 0.0)
                exch[e, pl.ds(0, L)] = ex
            return c2

        lax.fori_loop(0, CH // L, edge16, 0)
        pltpu.sync_copy(exch, zacc.at[dsti], add=True)
        pltpu.sync_copy(exch.at[pl.ds(0, CH), pl.ds(0, L)],
                        exb.at[pl.ds(base, CH)])
        return carry

    lax.fori_loop(0, CPW, chunk, 0)
    plsc.subcore_barrier()
    _acc_out(zacc, out, tid, cid)


def _gat_logit_pass(als16, ald16, src, dst, z32):
    f = pl.kernel(
        _gat_logit_body,
        out_type=[
            jax.ShapeDtypeStruct((NC * N, 32), jnp.float32),
            jax.ShapeDtypeStruct((EP, L), jnp.float32),
        ],
        mesh=_vsc_mesh(),
        compiler_params=_SC_PARAMS,
        scratch_types=[
            pltpu.VMEM_SHARED((NA, 32), jnp.float32),
            pltpu.VMEM((CH,), jnp.int32),
            pltpu.VMEM((CH,), jnp.int32),
            pltpu.VMEM((CH, L), jnp.float32),
            pltpu.VMEM((CH, L), jnp.float32),
            pltpu.VMEM((CH, 32), jnp.float32),
        ],
    )
    return f(als16, ald16, src, dst, z32)


def _gat_out_body(hsc, srch, dsth, exbh, zinvh, z32, out, acc,
                  srci0, srci1, dsti0, dsti1, exch0, exch1, zrows0, zrows1,
                  hrows0, hrows1, orows, sg0, sg1):
    cid = lax.axis_index("c")
    tid = lax.axis_index("s")
    _acc_zero(z32, acc, tid)
    plsc.subcore_barrier()
    off = cid * N
    base0 = tid * CPTO
    srcis = (srci0, srci1)
    dstis = (dsti0, dsti1)
    exchs = (exch0, exch1)
    zrowss = (zrows0, zrows1)
    hrowss = (hrows0, hrows1)
    sgs = (sg0, sg1)

    def load_idx(b, c):
        base = c * CHO
        pltpu.sync_copy(srch.at[pl.ds(base, CHO)], srcis[b])
        pltpu.sync_copy(dsth.at[pl.ds(base, CHO)], dstis[b])
        pltpu.sync_copy(exbh.at[pl.ds(base, CHO)], exchs[b])
        for i in range(CHO // L):
            srcis[b][pl.ds(i * L, L)] = srcis[b][pl.ds(i * L, L)] + off

    def issue(b):
        pltpu.async_copy(hsc.at[srcis[b]], hrowss[b], sgs[b])
        pltpu.async_copy(zinvh.at[dstis[b]], zrowss[b], sgs[b])

    def wait(b):
        pltpu.make_async_copy(hsc.at[srcis[b]], hrowss[b], sgs[b]).wait()
        pltpu.make_async_copy(zinvh.at[dstis[b]], zrowss[b], sgs[b]).wait()

    def compute(b):
        exch, zrows, hrows, dsti = exchs[b], zrowss[b], hrowss[b], dstis[b]

        def edge16(g, c2):
            for e2 in range(L):
                e = g * L + e2
                av = exch[e, pl.ds(0, L)] * zrows[e, pl.ds(0, L)]
                o0 = lax.broadcast_in_dim(av[0], (L,), ()) * hrows[e, pl.ds(0, L)]
                o1 = lax.broadcast_in_dim(av[0], (L,), ()) * hrows[e, pl.ds(L, L)]
                for h in range(1, 4):
                    ab = lax.broadcast_in_dim(av[h], (L,), ())
                    o0 = o0 + ab * hrows[e, pl.ds(h * 32, L)]
                    o1 = o1 + ab * hrows[e, pl.ds(h * 32 + L, L)]
                orows[e, pl.ds(0, L)] = o0
                orows[e, pl.ds(L, L)] = o1
            return c2

        lax.fori_loop(0, CHO // L, edge16, 0)
        pltpu.sync_copy(orows, acc.at[dsti], add=True)

    load_idx(0, base0)
    issue(0)

    def pair(j2, carry):
        c = base0 + 2 * j2
        load_idx(1, c + 1)
        wait(0)
        issue(1)
        compute(0)

        @pl.when(j2 < CPTO // 2 - 1)
        def _():
            load_idx(0, c + 2)
        wait(1)

        @pl.when(j2 < CPTO // 2 - 1)
        def _():
            issue(0)
        compute(1)
        return carry

    lax.fori_loop(0, CPTO // 2, pair, 0)
    plsc.subcore_barrier()
    _acc_out(acc, out, tid, cid)


def _gat_out_pass(hsc, src, dst, exb, zinv16, z32):
    f = pl.kernel(
        _gat_out_body,
        out_type=jax.ShapeDtypeStruct((NC * N, 32), jnp.float32),
        mesh=_vsc_mesh(),
        compiler_params=_SC_PARAMS,
        scratch_types=[
            pltpu.VMEM_SHARED((NA, 32), jnp.float32),
            pltpu.VMEM((CHO,), jnp.int32),
            pltpu.VMEM((CHO,), jnp.int32),
            pltpu.VMEM((CHO,), jnp.int32),
            pltpu.VMEM((CHO,), jnp.int32),
            pltpu.VMEM((CHO, L), jnp.float32),
            pltpu.VMEM((CHO, L), jnp.float32),
            pltpu.VMEM((CHO, L), jnp.float32),
            pltpu.VMEM((CHO, L), jnp.float32),
            pltpu.VMEM((CHO, 128), jnp.float32),
            pltpu.VMEM((CHO, 128), jnp.float32),
            pltpu.VMEM((CHO, 32), jnp.float32),
            pltpu.SemaphoreType.DMA,
            pltpu.SemaphoreType.DMA,
        ],
    )
    return f(hsc, src, dst, exb, zinv16, z32)


RCH = 400


def _pool_body(xh, bh, out, acc, xrows, bidx):
    cid = lax.axis_index("c")
    tid = lax.axis_index("s")
    wid = tid * NC + cid
    negv = jnp.full((L,), NEG, jnp.float32)

    def initrow(r, c):
        for k in range(HID // L):
            acc[r, pl.ds(k * L, L)] = negv
        return c

    lax.fori_loop(0, GRAPHS, initrow, 0)

    nloop = (N // RCH) // (NC * NT) + 1

    def chunk(j, carry):
        k = j * (NC * NT) + wid

        @pl.when(k < N // RCH)
        def _():
            base = k * RCH
            pltpu.sync_copy(xh.at[pl.ds(base, RCH)], xrows)
            pltpu.sync_copy(bh.at[pl.ds(base, RCH)], bidx)

            def row16(g, c2):
                bvec = bidx[pl.ds(g * L, L)]
                for r2 in range(L):
                    gid = bvec[r2]
                    r = g * L + r2
                    for k2 in range(HID // L):
                        cur = acc[gid, pl.ds(k2 * L, L)]
                        xv = xrows[r, pl.ds(k2 * L, L)]
                        acc[gid, pl.ds(k2 * L, L)] = jnp.maximum(cur, xv)
                return c2

            lax.fori_loop(0, RCH // L, row16, 0)
        return carry

    lax.fori_loop(0, nloop, chunk, 0)
    pltpu.sync_copy(acc, out.at[pl.ds(wid * GRAPHS, GRAPHS)])


def _pool_pass(xcur, batch):
    f = pl.kernel(
        _pool_body,
        out_type=jax.ShapeDtypeStruct((NC * NT * GRAPHS, HID), jnp.float32),
        mesh=_vsc_mesh(),
        compiler_params=_SC_PARAMS,
        scratch_types=[
            pltpu.VMEM((GRAPHS, HID), jnp.float32),
            pltpu.VMEM((RCH, HID), jnp.float32),
            pltpu.VMEM((RCH,), jnp.int32),
        ],
    )
    return f(xcur, batch)


def _tc2_body(x_ref, w_ref, degp_ref, hp_ref, dinv_ref):
    h1 = _mm(x_ref[...], w_ref[...])
    deg = 1.0 + degp_ref[0, :, 0] + degp_ref[1, :, 0]
    dinv = jnp.where(deg > 0, lax.rsqrt(deg), 0.0)
    hp = h1 * dinv[:, None]
    hp_ref[0] = hp[:, :32]
    hp_ref[1] = hp[:, 32:]
    dinv_ref[...] = jnp.broadcast_to(dinv[:, None], (BN, 8))


def _tc2(x, W_gcn, degp):
    return pl.pallas_call(
        _tc2_body,
        grid=(N // BN,),
        in_specs=[
            pl.BlockSpec((BN, F_IN), lambda i: (i, 0)),
            pl.BlockSpec((F_IN, HID), lambda i: (0, 0)),
            pl.BlockSpec((2, BN, L), lambda i: (0, i, 0)),
        ],
        out_specs=[
            pl.BlockSpec((2, BN, 32), lambda i: (0, i, 0)),
            pl.BlockSpec((BN, 8), lambda i: (i, 0)),
        ],
        out_shape=[
            jax.ShapeDtypeStruct((2, N, 32), jnp.float32),
            jax.ShapeDtypeStruct((N, 8), jnp.float32),
        ],
    )(x, W_gcn, degp)


def _gat_prep(H, asc_ref, adc_ref, g128, hsc_ref, als_ref, ald_ref):
    als = jnp.zeros((BN, 4), jnp.float32)
    ald = jnp.zeros((BN, 4), jnp.float32)
    for c in range(2):
        hc = jnp.concatenate(
            [H[:, h * 64 + c * 32:h * 64 + (c + 1) * 32] for h in range(4)],
            axis=1)
        hsc_ref[c] = hc
        als = als + _mm(hc * asc_ref[c][None, :], g128)
        ald = ald + _mm(hc * adc_ref[c][None, :], g128)
    pad = jnp.zeros((BN, 12), jnp.float32)
    als_ref[...] = jnp.concatenate([als, pad], axis=1)
    ald_ref[...] = jnp.concatenate([ald, pad], axis=1)


def _tc3_body(gacc_ref, hp_ref, dinv_ref, b_ref, gw_ref, asc_ref, adc_ref,
              g128_ref, xcur_ref, hsc_ref, als_ref, ald_ref):
    acc = jnp.concatenate([gacc_ref[0], gacc_ref[1]], axis=1)
    hpv = jnp.concatenate([hp_ref[0], hp_ref[1]], axis=1)
    dinv = dinv_ref[:, 0]
    xcur = jax.nn.relu((acc + hpv) * dinv[:, None] + b_ref[...])
    xcur_ref[...] = xcur
    H = _mm(xcur, gw_ref[...])
    _gat_prep(H, asc_ref, adc_ref, g128_ref[...], hsc_ref, als_ref, ald_ref)


def _tc3(gacc, hp, dinv8, b_gcn, gw, ascf, adcf, g128):
    return pl.pallas_call(
        _tc3_body,
        grid=(N // BN,),
        in_specs=[
            pl.BlockSpec((2, BN, 32), lambda i: (0, i, 0)),
            pl.BlockSpec((2, BN, 32), lambda i: (0, i, 0)),
            pl.BlockSpec((BN, 8), lambda i: (i, 0)),
            pl.BlockSpec((1, HID), lambda i: (0, 0)),
            pl.BlockSpec((HID, HEADS * HID), lambda i: (0, 0)),
            pl.BlockSpec((2, 128), lambda i: (0, 0)),
            pl.BlockSpec((2, 128), lambda i: (0, 0)),
            pl.BlockSpec((128, 4), lambda i: (0, 0)),
        ],
        out_specs=[
            pl.BlockSpec((BN, HID), lambda i: (i, 0)),
            pl.BlockSpec((2, BN, 128), lambda i: (0, i, 0)),
            pl.BlockSpec((BN, 16), lambda i: (i, 0)),
            pl.BlockSpec((BN, 16), lambda i: (i, 0)),
        ],
        out_shape=[
            jax.ShapeDtypeStruct((N, HID), jnp.float32),
            jax.ShapeDtypeStruct((2, N, 128), jnp.float32),
            jax.ShapeDtypeStruct((N, 16), jnp.float32),
            jax.ShapeDtypeStruct((N, 16), jnp.float32),
        ],
    )(gacc, hp, dinv8, b_gcn, gw, ascf, adcf, g128)


def _tc4_body(zp_ref, als_ref, ald_ref, zinv_ref, selfa_ref):
    al = als_ref[:, :4] + ald_ref[:, :4]
    ex_self = jnp.exp(jnp.where(al > 0, al, 0.2 * al))
    z = zp_ref[0, :, :4] + zp_ref[1, :, :4] + ex_self
    zinv = 1.0 / z
    pad = jnp.zeros((BN, 12), jnp.float32)
    zinv_ref[...] = jnp.concatenate([zinv, pad], axis=1)
    selfa_ref[...] = jnp.concatenate([ex_self * zinv, pad], axis=1)


def _tc4(zp, als16, ald16):
    return pl.pallas_call(
        _tc4_body,
        grid=(N // BN,),
        in_specs=[
            pl.BlockSpec((2, BN, 32), lambda i: (0, i, 0)),
            pl.BlockSpec((BN, 16), lambda i: (i, 0)),
            pl.BlockSpec((BN, 16), lambda i: (i, 0)),
        ],
        out_specs=[
            pl.BlockSpec((BN, 16), lambda i: (i, 0)),
            pl.BlockSpec((BN, 16), lambda i: (i, 0)),
        ],
        out_shape=[
            jax.ShapeDtypeStruct((N, 16), jnp.float32),
            jax.ShapeDtypeStruct((N, 16), jnp.float32),
        ],
    )(zp, als16, ald16)


def _tc5_layer1(accp, hsc, selfa16, xcur, g_b, g4, r128,
                fc1_W, fc1_b, fc2_W, fc2_b, pro_bias,
                gw2, asc2, adc2, g128):
    def body(accp_ref, hsc_ref, selfa_ref, xcur_ref, gb_ref, g4_ref, r128_ref,
             fc1w_ref, fc1b_ref, fc2w_ref, fc2b_ref, pro_ref,
             gw2_ref, asc2_ref, adc2_ref, g128_ref,
             xnew_ref, hsc2_ref, als2_ref, ald2_ref):
        selfexp = _mm(selfa_ref[...], g4_ref[...])
        halves = []
        for c in range(2):
            self_c = _mm(hsc_ref[c] * selfexp, r128_ref[...])
            halves.append(0.25 * (accp_ref[c] + self_c))
        xc = jax.nn.relu(jnp.concatenate(halves, axis=1) + gb_ref[...])
        xcur = xcur_ref[...]
        zg = jax.nn.sigmoid(_mm(xc, fc1w_ref[...]) + fc1b_ref[...]
                            + _mm(xcur, fc2w_ref[...]) + fc2b_ref[...] + pro_ref[...])
        xnew = zg * xc + (1.0 - zg) * xcur
        xnew_ref[...] = xnew
        H2 = _mm(xnew, gw2_ref[...])
        _gat_prep(H2, asc2_ref, adc2_ref, g128_ref[...], hsc2_ref, als2_ref, ald2_ref)

    return pl.pallas_call(
        body,
        grid=(N // BN,),
        in_specs=[
            pl.BlockSpec((2, BN, 32), lambda i: (0, i, 0)),
            pl.BlockSpec((2, BN, 128), lambda i: (0, i, 0)),
            pl.BlockSpec((BN, 16), lambda i: (i, 0)),
            pl.BlockSpec((BN, HID), lambda i: (i, 0)),
            pl.BlockSpec((1, HID), lambda i: (0, 0)),
            pl.BlockSpec((16, 128), lambda i: (0, 0)),
            pl.BlockSpec((128, 32), lambda i: (0, 0)),
            pl.BlockSpec((HID, HID), lambda i: (0, 0)),
            pl.BlockSpec((1, HID), lambda i: (0, 0)),
            pl.BlockSpec((HID, HID), lambda i: (0, 0)),
            pl.BlockSpec((1, HID), lambda i: (0, 0)),
            pl.BlockSpec((1, HID), lambda i: (0, 0)),
            pl.BlockSpec((HID, HEADS * HID), lambda i: (0, 0)),
            pl.BlockSpec((2, 128), lambda i: (0, 0)),
            pl.BlockSpec((2, 128), lambda i: (0, 0)),
            pl.BlockSpec((128, 4), lambda i: (0, 0)),
        ],
        out_specs=[
            pl.BlockSpec((BN, HID), lambda i: (i, 0)),
            pl.BlockSpec((2, BN, 128), lambda i: (0, i, 0)),
            pl.BlockSpec((BN, 16), lambda i: (i, 0)),
            pl.BlockSpec((BN, 16), lambda i: (i, 0)),
        ],
        out_shape=[
            jax.ShapeDtypeStruct((N, HID), jnp.float32),
            jax.ShapeDtypeStruct((2, N, 128), jnp.float32),
            jax.ShapeDtypeStruct((N, 16), jnp.float32),
            jax.ShapeDtypeStruct((N, 16), jnp.float32),
        ],
    )(accp, hsc, selfa16, xcur, g_b, g4, r128, fc1_W, fc1_b, fc2_W, fc2_b,
      pro_bias, gw2, asc2, adc2, g128)


def _tc5_layer2(accp, hsc, selfa16, xcur, g_b, g4, r128,
                fc1_W, fc1_b, fc2_W, fc2_b, pro_bias):
    def body(accp_ref, hsc_ref, selfa_ref, xcur_ref, gb_ref, g4_ref, r128_ref,
             fc1w_ref, fc1b_ref, fc2w_ref, fc2b_ref, pro_ref, xnew_ref):
        selfexp = _mm(selfa_ref[...], g4_ref[...])
        halves = []
        for c in range(2):
            self_c = _mm(hsc_ref[c] * selfexp, r128_ref[...])
            halves.append(0.25 * (accp_ref[c] + self_c))
        xc = jnp.concatenate(halves, axis=1) + gb_ref[...]
        xcur = xcur_ref[...]
        zg = jax.nn.sigmoid(_mm(xc, fc1w_ref[...]) + fc1b_ref[...]
                            + _mm(xcur, fc2w_ref[...]) + fc2b_ref[...] + pro_ref[...])
        xnew_ref[...] = zg * xc + (1.0 - zg) * xcur

    return pl.pallas_call(
        body,
        grid=(N // BN,),
        in_specs=[
            pl.BlockSpec((2, BN, 32), lambda i: (0, i, 0)),
            pl.BlockSpec((2, BN, 128), lambda i: (0, i, 0)),
            pl.BlockSpec((BN, 16), lambda i: (i, 0)),
            pl.BlockSpec((BN, HID), lambda i: (i, 0)),
            pl.BlockSpec((1, HID), lambda i: (0, 0)),
            pl.BlockSpec((16, 128), lambda i: (0, 0)),
            pl.BlockSpec((128, 32), lambda i: (0, 0)),
            pl.BlockSpec((HID, HID), lambda i: (0, 0)),
            pl.BlockSpec((1, HID), lambda i: (0, 0)),
            pl.BlockSpec((HID, HID), lambda i: (0, 0)),
            pl.BlockSpec((1, HID), lambda i: (0, 0)),
            pl.BlockSpec((1, HID), lambda i: (0, 0)),
        ],
        out_specs=[pl.BlockSpec((BN, HID), lambda i: (i, 0))],
        out_shape=[jax.ShapeDtypeStruct((N, HID), jnp.float32)],
    )(accp, hsc, selfa16, xcur, g_b, g4, r128, fc1_W, fc1_b, fc2_W, fc2_b,
      pro_bias)


def _mlp_head_body(pooled_parts_ref, w1_ref, b1_ref, w2_ref, b2_ref, out_ref):
    pooled = jnp.max(pooled_parts_ref[...], axis=0)
    pooled = jnp.where(pooled > -1.0e38, pooled, 0.0)
    h = jax.nn.relu(_mm(pooled, w1_ref[...]) + b1_ref[...])
    out_ref[...] = _mm(h, w2_ref[...]) + b2_ref[...]


def _mlp_head(pooled_parts, w1, b1, w2, b2):
    return pl.pallas_call(
        _mlp_head_body,
        out_shape=jax.ShapeDtypeStruct((GRAPHS, OUT_DIM), jnp.float32),
    )(pooled_parts, w1, b1[None], w2, b2[None])


def _prep_a(a):
    return a.reshape(4, 2, 32).transpose(1, 0, 2).reshape(2, 128)


def kernel(x, edge_index, edge_weight, batch, W_gcn, b_gcn, g1_W, g1_as, g1_ad, g1_b, g2_W, g2_as, g2_ad, g2_b, fc1_W, fc1_b, fc2_W, fc2_b, pro_bias, gfc1_W, gfc1_b, gfc2_W, gfc2_b):
    src = edge_index[0]
    dst = edge_index[1]
    npad = EP - E
    src = jnp.concatenate([src, jnp.zeros((npad,), jnp.int32)])
    dst = jnp.concatenate([dst, jnp.full((npad,), N, jnp.int32)])
    ew_p = jnp.concatenate([edge_weight, jnp.zeros((npad,), jnp.float32)])
    z16 = jnp.zeros((N, L), jnp.float32)
    z32 = jnp.zeros((N, 32), jnp.float32)

    def padn(a):
        return jnp.concatenate([a, jnp.zeros((8, a.shape[1]), a.dtype)])
    g128 = (jnp.arange(128)[:, None] // 32 == jnp.arange(4)[None, :]).astype(jnp.float32)
    g4 = (jnp.arange(16)[:, None] == jnp.arange(128)[None, :] // 32).astype(jnp.float32)
    r128 = (jnp.arange(128)[:, None] % 32 == jnp.arange(32)[None, :]).astype(jnp.float32)

    w16 = jnp.broadcast_to(ew_p[:, None], (EP, L))
    degp = _deg_pass(dst, w16, z16).reshape(2, N, L)
    hp2, dinv8 = _tc2(x, W_gcn, degp)
    gacc = _gcn_pass(hp2.reshape(2 * N, 32), src, dst, ew_p, z32)
    xcur, Hsc, als16, ald16 = _tc3(gacc.reshape(2, N, 32), hp2, dinv8,
                                   b_gcn[None], g1_W, _prep_a(g1_as),
                                   _prep_a(g1_ad), g128)

    zp, exb = _gat_logit_pass(padn(als16), padn(ald16), src, dst, z32)
    zinv16, selfa16 = _tc4(zp.reshape(2, N, 32), als16, ald16)
    acc1 = _gat_out_pass(Hsc.reshape(2 * N, 128), src, dst, exb, padn(zinv16), z32)
    xcur, Hsc2, als26, ald26 = _tc5_layer1(
        acc1.reshape(2, N, 32), Hsc, selfa16, xcur, g1_b[None], g4, r128,
        fc1_W, fc1_b[None], fc2_W, fc2_b[None], pro_bias[None],
        g2_W, _prep_a(g2_as), _prep_a(g2_ad), g128)

    zp2, exb2 = _gat_logit_pass(padn(als26), padn(ald26), src, dst, z32)
    zinv26, selfa26 = _tc4(zp2.reshape(2, N, 32), als26, ald26)
    acc2 = _gat_out_pass(Hsc2.reshape(2 * N, 128), src, dst, exb2, padn(zinv26), z32)
    (xcur,) = _tc5_layer2(
        acc2.reshape(2, N, 32), Hsc2, selfa26, xcur, g2_b[None], g4, r128,
        fc1_W, fc1_b[None], fc2_W, fc2_b[None], pro_bias[None])

    parts = _pool_pass(xcur, batch).reshape(NC * NT, GRAPHS, HID)
    return _mlp_head(parts, gfc1_W, gfc1_b, gfc2_W, gfc2_b)


# speedup vs baseline: 29.4070x; 1.0735x over previous
import functools

import jax
import jax.numpy as jnp
from jax import lax
from jax.experimental import pallas as pl
from jax.experimental.pallas import tpu as pltpu
from jax.experimental.pallas import tpu_sc as plsc

N = 50000
E = 800000
F_IN = 16
HID = 64
HEADS = 4
GRAPHS = 256
OUT_DIM = 128

NC = 2
NT = 16
L = 16
CH = 128
NCHUNK = E // CH
EP = 802816
EPC = EP // CH
CPT = EPC // NT
CPW = EPC // (NC * NT)
NA = N + 8
CHO = 64
CPTO = (EP // CHO) // NT
PER = 3128
LAST = N - PER * (NT - 1)
BN = 2000
NEG = -3.0e38


def _acc_zero(zsrc, acc, tid):
    @pl.when(tid < NT - 1)
    def _():
        o = pl.multiple_of(tid * PER, 8)
        pltpu.sync_copy(zsrc.at[pl.ds(o, PER)], acc.at[pl.ds(o, PER)])

    @pl.when(tid == NT - 1)
    def _():
        pltpu.sync_copy(zsrc.at[pl.ds(PER * (NT - 1), LAST)],
                        acc.at[pl.ds(PER * (NT - 1), LAST)])


def _acc_out(acc, out, tid, cid):
    @pl.when(tid < NT - 1)
    def _():
        o = pl.multiple_of(tid * PER, 8)
        oo = pl.multiple_of(cid * N + tid * PER, 8)
        pltpu.sync_copy(acc.at[pl.ds(o, PER)], out.at[pl.ds(oo, PER)])

    @pl.when(tid == NT - 1)
    def _():
        oo = pl.multiple_of(cid * N + PER * (NT - 1), 8)
        pltpu.sync_copy(acc.at[pl.ds(PER * (NT - 1), LAST)], out.at[pl.ds(oo, LAST)])


def _vsc_mesh():
    return plsc.VectorSubcoreMesh(
        core_axis_name="c", subcore_axis_name="s", num_cores=NC, num_subcores=NT)


_SC_PARAMS = pltpu.CompilerParams(use_tc_tiling_on_sc=False)

def _mm(a, b):
    return jax.lax.dot_general(a, b, (((a.ndim - 1,), (0,)), ((), ())),
                               precision=jax.lax.Precision.HIGHEST)


def _deg_body(dsth, w16h, z16, out, dacc, dsti, wrow):
    cid = lax.axis_index("c")
    tid = lax.axis_index("s")
    wid = tid * NC + cid
    _acc_zero(z16, dacc, tid)
    plsc.subcore_barrier()

    def chunk(j, carry):
        base = (j * (NC * NT) + wid) * CH
        pltpu.sync_copy(dsth.at[pl.ds(base, CH)], dsti)
        pltpu.sync_copy(w16h.at[pl.ds(base, CH)], wrow)
        pltpu.sync_copy(wrow, dacc.at[dsti], add=True)
        return carry

    lax.fori_loop(0, CPW, chunk, 0)
    plsc.subcore_barrier()
    _acc_out(dacc, out, tid, cid)


def _deg_pass(dst, w16, z16):
    f = pl.kernel(
        _deg_body,
        out_type=jax.ShapeDtypeStruct((NC * N, L), jnp.float32),
        mesh=_vsc_mesh(),
        compiler_params=_SC_PARAMS,
        scratch_types=[
            pltpu.VMEM_SHARED((NA, L), jnp.float32),
            pltpu.VMEM((CH,), jnp.int32),
            pltpu.VMEM((CH, L), jnp.float32),
        ],
    )
    return f(dst, w16, z16)


def _gcn_body(hp, srch, dsth, wh, z32, out, acc,
              srci0, srci1, dsti0, dsti1, wv0, wv1, rows0, rows1, sg0, sg1):
    cid = lax.axis_index("c")
    tid = lax.axis_index("s")
    _acc_zero(z32, acc, tid)
    plsc.subcore_barrier()
    off = cid * N
    base0 = tid * CPT
    srcis = (srci0, srci1)
    dstis = (dsti0, dsti1)
    wvs = (wv0, wv1)
    rowss = (rows0, rows1)
    sgs = (sg0, sg1)

    def load_idx(b, c):
        base = c * CH
        pltpu.sync_copy(srch.at[pl.ds(base, CH)], srcis[b])
        pltpu.sync_copy(dsth.at[pl.ds(base, CH)], dstis[b])
        pltpu.sync_copy(wh.at[pl.ds(base, CH)], wvs[b])
        for i in range(CH // L):
            srcis[b][pl.ds(i * L, L)] = srcis[b][pl.ds(i * L, L)] + off

    def issue(b):
        pltpu.async_copy(hp.at[srcis[b]], rowss[b], sgs[b])

    def wait(b):
        pltpu.make_async_copy(hp.at[srcis[b]], rowss[b], sgs[b]).wait()

    def compute(b):
        rows, wv, dsti = rowss[b], wvs[b], dstis[b]

        def group(g, c2):
            wvec = wv[pl.ds(g * L, L)]
            for e2 in range(L):
                wb = lax.broadcast_in_dim(wvec[e2], (L,), ())
                e = g * L + e2
                for k2 in range(2):
                    v = rows[e, pl.ds(k2 * L, L)]
                    rows[e, pl.ds(k2 * L, L)] = v * wb
            return c2

        lax.fori_loop(0, CH // L, group, 0)
        pltpu.sync_copy(rows, acc.at[dsti], add=True)

    load_idx(0, base0)
    issue(0)

    def pair(j2, carry):
        c = base0 + 2 * j2
        load_idx(1, c + 1)
        wait(0)
        issue(1)
        compute(0)
        @pl.when(j2 < CPT // 2 - 1)
        def _():
            load_idx(0, c + 2)
        wait(1)

        @pl.when(j2 < CPT // 2 - 1)
        def _():
            issue(0)
        compute(1)
        return carry

    lax.fori_loop(0, CPT // 2, pair, 0)
    plsc.subcore_barrier()
    _acc_out(acc, out, tid, cid)


def _gcn_pass(hp, src, dst, w, z32):
    f = pl.kernel(
        _gcn_body,
        out_type=jax.ShapeDtypeStruct((NC * N, 32), jnp.float32),
        mesh=_vsc_mesh(),
        compiler_params=_SC_PARAMS,
        scratch_types=[
            pltpu.VMEM_SHARED((NA, 32), jnp.float32),
            pltpu.VMEM((CH,), jnp.int32),
            pltpu.VMEM((CH,), jnp.int32),
            pltpu.VMEM((CH,), jnp.int32),
            pltpu.VMEM((CH,), jnp.int32),
            pltpu.VMEM((CH,), jnp.float32),
            pltpu.VMEM((CH,), jnp.float32),
            pltpu.VMEM((CH, 32), jnp.float32),
            pltpu.VMEM((CH, 32), jnp.float32),
            pltpu.SemaphoreType.DMA,
            pltpu.SemaphoreType.DMA,
        ],
    )
    return f(hp, src, dst, w, z32)


def _gat_logit_body(alsh, aldh, srch, dsth, z32, out, exb, zacc,
                    srci0, srci1, dsti0, dsti1, asrc0, asrc1, adst0, adst1,
                    exch, sg0, sg1):
    cid = lax.axis_index("c")
    tid = lax.axis_index("s")
    wid = tid * NC + cid
    iota = lax.iota(jnp.int32, L)
    lanelt4 = iota < 4
    _acc_zero(z32, zacc, tid)
    pltpu.sync_copy(z32.at[pl.ds(0, CH)], exch)
    plsc.subcore_barrier()
    base0 = wid * CPW
    srcis = (srci0, srci1)
    dstis = (dsti0, dsti1)
    asrcs = (asrc0, asrc1)
    adsts = (adst0, adst1)
    sgs = (sg0, sg1)

    def load_idx(b, c):
        base = c * CH
        pltpu.sync_copy(srch.at[pl.ds(base, CH)], srcis[b])
        pltpu.sync_copy(dsth.at[pl.ds(base, CH)], dstis[b])

    def issue(b):
        pltpu.async_copy(alsh.at[srcis[b]], asrcs[b], sgs[b])
        pltpu.async_copy(aldh.at[dstis[b]], adsts[b], sgs[b])

    def wait(b):
        pltpu.make_async_copy(alsh.at[srcis[b]], asrcs[b], sgs[b]).wait()
        pltpu.make_async_copy(aldh.at[dstis[b]], adsts[b], sgs[b]).wait()

    def compute(b, c):
        asrc, adst, dsti = asrcs[b], adsts[b], dstis[b]

        def edge16(g, c2):
            for e2 in range(L):
                e = g * L + e2
                ev = asrc[e, pl.ds(0, L)] + adst[e, pl.ds(0, L)]
                ev = jnp.where(ev > 0, ev, 0.2 * ev)
                ex = jnp.where(lanelt4, jnp.exp(ev), 0.0)
                exch[e, pl.ds(0, L)] = ex
            return c2

        lax.fori_loop(0, CH // L, edge16, 0)
        pltpu.sync_copy(exch, zacc.at[dsti], add=True)
        pltpu.sync_copy(exch.at[pl.ds(0, CH), pl.ds(0, L)],
                        exb.at[pl.ds(c * CH, CH)])

    load_idx(0, base0)
    issue(0)

    def pair(j2, carry):
        c = base0 + 2 * j2
        load_idx(1, c + 1)
        wait(0)
        issue(1)
        compute(0, c)

        @pl.when(j2 < CPW // 2 - 1)
        def _():
            load_idx(0, c + 2)
        wait(1)

        @pl.when(j2 < CPW // 2 - 1)
        def _():
            issue(0)
        compute(1, c + 1)
        return carry

    lax.fori_loop(0, CPW // 2, pair, 0)
    plsc.subcore_barrier()
    _acc_out(zacc, out, tid, cid)


def _gat_logit_pass(als16, ald16, src, dst, z32):
    f = pl.kernel(
        _gat_logit_body,
        out_type=[
            jax.ShapeDtypeStruct((NC * N, 32), jnp.float32),
            jax.ShapeDtypeStruct((EP, L), jnp.float32),
        ],
        mesh=_vsc_mesh(),
        compiler_params=_SC_PARAMS,
        scratch_types=[
            pltpu.VMEM_SHARED((NA, 32), jnp.float32),
            pltpu.VMEM((CH,), jnp.int32),
            pltpu.VMEM((CH,), jnp.int32),
            pltpu.VMEM((CH,), jnp.int32),
            pltpu.VMEM((CH,), jnp.int32),
            pltpu.VMEM((CH, L), jnp.float32),
            pltpu.VMEM((CH, L), jnp.float32),
            pltpu.VMEM((CH, L), jnp.float32),
            pltpu.VMEM((CH, L), jnp.float32),
            pltpu.VMEM((CH, 32), jnp.float32),
            pltpu.SemaphoreType.DMA,
            pltpu.SemaphoreType.DMA,
        ],
    )
    return f(als16, ald16, src, dst, z32)


def _gat_out_body(hsc, srch, dsth, exbh, zinvh, z32, out, acc,
                  srci0, srci1, dsti0, dsti1, exch0, exch1, zrows0, zrows1,
                  hrows0, hrows1, orows, sg0, sg1):
    cid = lax.axis_index("c")
    tid = lax.axis_index("s")
    _acc_zero(z32, acc, tid)
    plsc.subcore_barrier()
    off = cid * N
    base0 = tid * CPTO
    srcis = (srci0, srci1)
    dstis = (dsti0, dsti1)
    exchs = (exch0, exch1)
    zrowss = (zrows0, zrows1)
    hrowss = (hrows0, hrows1)
    sgs = (sg0, sg1)

    def load_idx(b, c):
        base = c * CHO
        pltpu.sync_copy(srch.at[pl.ds(base, CHO)], srcis[b])
        pltpu.sync_copy(dsth.at[pl.ds(base, CHO)], dstis[b])
        pltpu.sync_copy(exbh.at[pl.ds(base, CHO)], exchs[b])
        for i in range(CHO // L):
            srcis[b][pl.ds(i * L, L)] = srcis[b][pl.ds(i * L, L)] + off

    def issue(b):
        pltpu.async_copy(hsc.at[srcis[b]], hrowss[b], sgs[b])
        pltpu.async_copy(zinvh.at[dstis[b]], zrowss[b], sgs[b])

    def wait(b):
        pltpu.make_async_copy(hsc.at[srcis[b]], hrowss[b], sgs[b]).wait()
        pltpu.make_async_copy(zinvh.at[dstis[b]], zrowss[b], sgs[b]).wait()

    def compute(b):
        exch, zrows, hrows, dsti = exchs[b], zrowss[b], hrowss[b], dstis[b]

        def edge16(g, c2):
            for e2 in range(L):
                e = g * L + e2
                av = exch[e, pl.ds(0, L)] * zrows[e, pl.ds(0, L)]
                o0 = lax.broadcast_in_dim(av[0], (L,), ()) * hrows[e, pl.ds(0, L)]
                o1 = lax.broadcast_in_dim(av[0], (L,), ()) * hrows[e, pl.ds(L, L)]
                for h in range(1, 4):
                    ab = lax.broadcast_in_dim(av[h], (L,), ())
                    o0 = o0 + ab * hrows[e, pl.ds(h * 32, L)]
                    o1 = o1 + ab * hrows[e, pl.ds(h * 32 + L, L)]
                orows[e, pl.ds(0, L)] = o0
                orows[e, pl.ds(L, L)] = o1
            return c2

        lax.fori_loop(0, CHO // L, edge16, 0)
        pltpu.sync_copy(orows, acc.at[dsti], add=True)

    load_idx(0, base0)
    issue(0)

    def pair(j2, carry):
        c = base0 + 2 * j2
        load_idx(1, c + 1)
        wait(0)
        issue(1)
        compute(0)

        @pl.when(j2 < CPTO // 2 - 1)
        def _():
            load_idx(0, c + 2)
        wait(1)

        @pl.when(j2 < CPTO // 2 - 1)
        def _():
            issue(0)
        compute(1)
        return carry

    lax.fori_loop(0, CPTO // 2, pair, 0)
    plsc.subcore_barrier()
    _acc_out(acc, out, tid, cid)


def _gat_out_pass(hsc, src, dst, exb, zinv16, z32):
    f = pl.kernel(
        _gat_out_body,
        out_type=jax.ShapeDtypeStruct((NC * N, 32), jnp.float32),
        mesh=_vsc_mesh(),
        compiler_params=_SC_PARAMS,
        scratch_types=[
            pltpu.VMEM_SHARED((NA, 32), jnp.float32),
            pltpu.VMEM((CHO,), jnp.int32),
            pltpu.VMEM((CHO,), jnp.int32),
            pltpu.VMEM((CHO,), jnp.int32),
            pltpu.VMEM((CHO,), jnp.int32),
            pltpu.VMEM((CHO, L), jnp.float32),
            pltpu.VMEM((CHO, L), jnp.float32),
            pltpu.VMEM((CHO, L), jnp.float32),
            pltpu.VMEM((CHO, L), jnp.float32),
            pltpu.VMEM((CHO, 128), jnp.float32),
            pltpu.VMEM((CHO, 128), jnp.float32),
            pltpu.VMEM((CHO, 32), jnp.float32),
            pltpu.SemaphoreType.DMA,
            pltpu.SemaphoreType.DMA,
        ],
    )
    return f(hsc, src, dst, exb, zinv16, z32)


RCH = 400


def _pool_body(xh, bh, out, acc, xrows, bidx):
    cid = lax.axis_index("c")
    tid = lax.axis_index("s")
    wid = tid * NC + cid
    negv = jnp.full((L,), NEG, jnp.float32)

    def initrow(r, c):
        for k in range(HID // L):
            acc[r, pl.ds(k * L, L)] = negv
        return c

    lax.fori_loop(0, GRAPHS, initrow, 0)

    nloop = (N // RCH) // (NC * NT) + 1

    def chunk(j, carry):
        k = j * (NC * NT) + wid

        @pl.when(k < N // RCH)
        def _():
            base = k * RCH
            pltpu.sync_copy(xh.at[pl.ds(base, RCH)], xrows)
            pltpu.sync_copy(bh.at[pl.ds(base, RCH)], bidx)

            def row16(g, c2):
                bvec = bidx[pl.ds(g * L, L)]
                for r2 in range(L):
                    gid = bvec[r2]
                    r = g * L + r2
                    for k2 in range(HID // L):
                        cur = acc[gid, pl.ds(k2 * L, L)]
                        xv = xrows[r, pl.ds(k2 * L, L)]
                        acc[gid, pl.ds(k2 * L, L)] = jnp.maximum(cur, xv)
                return c2

            lax.fori_loop(0, RCH // L, row16, 0)
        return carry

    lax.fori_loop(0, nloop, chunk, 0)
    pltpu.sync_copy(acc, out.at[pl.ds(wid * GRAPHS, GRAPHS)])


def _pool_pass(xcur, batch):
    f = pl.kernel(
        _pool_body,
        out_type=jax.ShapeDtypeStruct((NC * NT * GRAPHS, HID), jnp.float32),
        mesh=_vsc_mesh(),
        compiler_params=_SC_PARAMS,
        scratch_types=[
            pltpu.VMEM((GRAPHS, HID), jnp.float32),
            pltpu.VMEM((RCH, HID), jnp.float32),
            pltpu.VMEM((RCH,), jnp.int32),
        ],
    )
    return f(xcur, batch)


def _tc2_body(x_ref, w_ref, degp_ref, hp_ref, dinv_ref):
    h1 = _mm(x_ref[...], w_ref[...])
    deg = 1.0 + degp_ref[0, :, 0] + degp_ref[1, :, 0]
    dinv = jnp.where(deg > 0, lax.rsqrt(deg), 0.0)
    hp = h1 * dinv[:, None]
    hp_ref[0] = hp[:, :32]
    hp_ref[1] = hp[:, 32:]
    dinv_ref[...] = jnp.broadcast_to(dinv[:, None], (BN, 8))


def _tc2(x, W_gcn, degp):
    return pl.pallas_call(
        _tc2_body,
        grid=(N // BN,),
        in_specs=[
            pl.BlockSpec((BN, F_IN), lambda i: (i, 0)),
            pl.BlockSpec((F_IN, HID), lambda i: (0, 0)),
            pl.BlockSpec((2, BN, L), lambda i: (0, i, 0)),
        ],
        out_specs=[
            pl.BlockSpec((2, BN, 32), lambda i: (0, i, 0)),
            pl.BlockSpec((BN, 8), lambda i: (i, 0)),
        ],
        out_shape=[
            jax.ShapeDtypeStruct((2, N, 32), jnp.float32),
            jax.ShapeDtypeStruct((N, 8), jnp.float32),
        ],
    )(x, W_gcn, degp)


def _gat_prep(H, asc_ref, adc_ref, g128, hsc_ref, als_ref, ald_ref):
    als = jnp.zeros((BN, 4), jnp.float32)
    ald = jnp.zeros((BN, 4), jnp.float32)
    for c in range(2):
        hc = jnp.concatenate(
            [H[:, h * 64 + c * 32:h * 64 + (c + 1) * 32] for h in range(4)],
            axis=1)
        hsc_ref[c] = hc
        als = als + _mm(hc * asc_ref[c][None, :], g128)
        ald = ald + _mm(hc * adc_ref[c][None, :], g128)
    pad = jnp.zeros((BN, 12), jnp.float32)
    als_ref[...] = jnp.concatenate([als, pad], axis=1)
    ald_ref[...] = jnp.concatenate([ald, pad], axis=1)


def _tc3_body(gacc_ref, hp_ref, dinv_ref, b_ref, gw_ref, asc_ref, adc_ref,
              g128_ref, xcur_ref, hsc_ref, als_ref, ald_ref):
    acc = jnp.concatenate([gacc_ref[0], gacc_ref[1]], axis=1)
    hpv = jnp.concatenate([hp_ref[0], hp_ref[1]], axis=1)
    dinv = dinv_ref[:, 0]
    xcur = jax.nn.relu((acc + hpv) * dinv[:, None] + b_ref[...])
    xcur_ref[...] = xcur
    H = _mm(xcur, gw_ref[...])
    _gat_prep(H, asc_ref, adc_ref, g128_ref[...], hsc_ref, als_ref, ald_ref)


def _tc3(gacc, hp, dinv8, b_gcn, gw, ascf, adcf, g128):
    return pl.pallas_call(
        _tc3_body,
        grid=(N // BN,),
        in_specs=[
            pl.BlockSpec((2, BN, 32), lambda i: (0, i, 0)),
            pl.BlockSpec((2, BN, 32), lambda i: (0, i, 0)),
            pl.BlockSpec((BN, 8), lambda i: (i, 0)),
            pl.BlockSpec((1, HID), lambda i: (0, 0)),
            pl.BlockSpec((HID, HEADS * HID), lambda i: (0, 0)),
            pl.BlockSpec((2, 128), lambda i: (0, 0)),
            pl.BlockSpec((2, 128), lambda i: (0, 0)),
            pl.BlockSpec((128, 4), lambda i: (0, 0)),
        ],
        out_specs=[
            pl.BlockSpec((BN, HID), lambda i: (i, 0)),
            pl.BlockSpec((2, BN, 128), lambda i: (0, i, 0)),
            pl.BlockSpec((BN, 16), lambda i: (i, 0)),
            pl.BlockSpec((BN, 16), lambda i: (i, 0)),
        ],
        out_shape=[
            jax.ShapeDtypeStruct((N, HID), jnp.float32),
            jax.ShapeDtypeStruct((2, N, 128), jnp.float32),
            jax.ShapeDtypeStruct((N, 16), jnp.float32),
            jax.ShapeDtypeStruct((N, 16), jnp.float32),
        ],
    )(gacc, hp, dinv8, b_gcn, gw, ascf, adcf, g128)


def _tc4_body(zp_ref, als_ref, ald_ref, zinv_ref, selfa_ref):
    al = als_ref[:, :4] + ald_ref[:, :4]
    ex_self = jnp.exp(jnp.where(al > 0, al, 0.2 * al))
    z = zp_ref[0, :, :4] + zp_ref[1, :, :4] + ex_self
    zinv = 1.0 / z
    pad = jnp.zeros((BN, 12), jnp.float32)
    zinv_ref[...] = jnp.concatenate([zinv, pad], axis=1)
    selfa_ref[...] = jnp.concatenate([ex_self * zinv, pad], axis=1)


def _tc4(zp, als16, ald16):
    return pl.pallas_call(
        _tc4_body,
        grid=(N // BN,),
        in_specs=[
            pl.BlockSpec((2, BN, 32), lambda i: (0, i, 0)),
            pl.BlockSpec((BN, 16), lambda i: (i, 0)),
            pl.BlockSpec((BN, 16), lambda i: (i, 0)),
        ],
        out_specs=[
            pl.BlockSpec((BN, 16), lambda i: (i, 0)),
            pl.BlockSpec((BN, 16), lambda i: (i, 0)),
        ],
        out_shape=[
            jax.ShapeDtypeStruct((N, 16), jnp.float32),
            jax.ShapeDtypeStruct((N, 16), jnp.float32),
        ],
    )(zp, als16, ald16)


def _tc5_layer1(accp, hsc, selfa16, xcur, g_b, g4, r128,
                fc1_W, fc1_b, fc2_W, fc2_b, pro_bias,
                gw2, asc2, adc2, g128):
    def body(accp_ref, hsc_ref, selfa_ref, xcur_ref, gb_ref, g4_ref, r128_ref,
             fc1w_ref, fc1b_ref, fc2w_ref, fc2b_ref, pro_ref,
             gw2_ref, asc2_ref, adc2_ref, g128_ref,
             xnew_ref, hsc2_ref, als2_ref, ald2_ref):
        selfexp = _mm(selfa_ref[...], g4_ref[...])
        halves = []
        for c in range(2):
            self_c = _mm(hsc_ref[c] * selfexp, r128_ref[...])
            halves.append(0.25 * (accp_ref[c] + self_c))
        xc = jax.nn.relu(jnp.concatenate(halves, axis=1) + gb_ref[...])
        xcur = xcur_ref[...]
        zg = jax.nn.sigmoid(_mm(xc, fc1w_ref[...]) + fc1b_ref[...]
                            + _mm(xcur, fc2w_ref[...]) + fc2b_ref[...] + pro_ref[...])
        xnew = zg * xc + (1.0 - zg) * xcur
        xnew_ref[...] = xnew
        H2 = _mm(xnew, gw2_ref[...])
        _gat_prep(H2, asc2_ref, adc2_ref, g128_ref[...], hsc2_ref, als2_ref, ald2_ref)

    return pl.pallas_call(
        body,
        grid=(N // BN,),
        in_specs=[
            pl.BlockSpec((2, BN, 32), lambda i: (0, i, 0)),
            pl.BlockSpec((2, BN, 128), lambda i: (0, i, 0)),
            pl.BlockSpec((BN, 16), lambda i: (i, 0)),
            pl.BlockSpec((BN, HID), lambda i: (i, 0)),
            pl.BlockSpec((1, HID), lambda i: (0, 0)),
            pl.BlockSpec((16, 128), lambda i: (0, 0)),
            pl.BlockSpec((128, 32), lambda i: (0, 0)),
            pl.BlockSpec((HID, HID), lambda i: (0, 0)),
            pl.BlockSpec((1, HID), lambda i: (0, 0)),
            pl.BlockSpec((HID, HID), lambda i: (0, 0)),
            pl.BlockSpec((1, HID), lambda i: (0, 0)),
            pl.BlockSpec((1, HID), lambda i: (0, 0)),
            pl.BlockSpec((HID, HEADS * HID), lambda i: (0, 0)),
            pl.BlockSpec((2, 128), lambda i: (0, 0)),
            pl.BlockSpec((2, 128), lambda i: (0, 0)),
            pl.BlockSpec((128, 4), lambda i: (0, 0)),
        ],
        out_specs=[
            pl.BlockSpec((BN, HID), lambda i: (i, 0)),
            pl.BlockSpec((2, BN, 128), lambda i: (0, i, 0)),
            pl.BlockSpec((BN, 16), lambda i: (i, 0)),
            pl.BlockSpec((BN, 16), lambda i: (i, 0)),
        ],
        out_shape=[
            jax.ShapeDtypeStruct((N, HID), jnp.float32),
            jax.ShapeDtypeStruct((2, N, 128), jnp.float32),
            jax.ShapeDtypeStruct((N, 16), jnp.float32),
            jax.ShapeDtypeStruct((N, 16), jnp.float32),
        ],
    )(accp, hsc, selfa16, xcur, g_b, g4, r128, fc1_W, fc1_b, fc2_W, fc2_b,
      pro_bias, gw2, asc2, adc2, g128)


def _tc5_layer2(accp, hsc, selfa16, xcur, g_b, g4, r128,
                fc1_W, fc1_b, fc2_W, fc2_b, pro_bias):
    def body(accp_ref, hsc_ref, selfa_ref, xcur_ref, gb_ref, g4_ref, r128_ref,
             fc1w_ref, fc1b_ref, fc2w_ref, fc2b_ref, pro_ref, xnew_ref):
        selfexp = _mm(selfa_ref[...], g4_ref[...])
        halves = []
        for c in range(2):
            self_c = _mm(hsc_ref[c] * selfexp, r128_ref[...])
            halves.append(0.25 * (accp_ref[c] + self_c))
        xc = jnp.concatenate(halves, axis=1) + gb_ref[...]
        xcur = xcur_ref[...]
        zg = jax.nn.sigmoid(_mm(xc, fc1w_ref[...]) + fc1b_ref[...]
                            + _mm(xcur, fc2w_ref[...]) + fc2b_ref[...] + pro_ref[...])
        xnew_ref[...] = zg * xc + (1.0 - zg) * xcur

    return pl.pallas_call(
        body,
        grid=(N // BN,),
        in_specs=[
            pl.BlockSpec((2, BN, 32), lambda i: (0, i, 0)),
            pl.BlockSpec((2, BN, 128), lambda i: (0, i, 0)),
            pl.BlockSpec((BN, 16), lambda i: (i, 0)),
            pl.BlockSpec((BN, HID), lambda i: (i, 0)),
            pl.BlockSpec((1, HID), lambda i: (0, 0)),
            pl.BlockSpec((16, 128), lambda i: (0, 0)),
            pl.BlockSpec((128, 32), lambda i: (0, 0)),
            pl.BlockSpec((HID, HID), lambda i: (0, 0)),
            pl.BlockSpec((1, HID), lambda i: (0, 0)),
            pl.BlockSpec((HID, HID), lambda i: (0, 0)),
            pl.BlockSpec((1, HID), lambda i: (0, 0)),
            pl.BlockSpec((1, HID), lambda i: (0, 0)),
        ],
        out_specs=[pl.BlockSpec((BN, HID), lambda i: (i, 0))],
        out_shape=[jax.ShapeDtypeStruct((N, HID), jnp.float32)],
    )(accp, hsc, selfa16, xcur, g_b, g4, r128, fc1_W, fc1_b, fc2_W, fc2_b,
      pro_bias)


def _mlp_head_body(pooled_parts_ref, w1_ref, b1_ref, w2_ref, b2_ref, out_ref):
    pooled = jnp.max(pooled_parts_ref[...], axis=0)
    pooled = jnp.where(pooled > -1.0e38, pooled, 0.0)
    h = jax.nn.relu(_mm(pooled, w1_ref[...]) + b1_ref[...])
    out_ref[...] = _mm(h, w2_ref[...]) + b2_ref[...]


def _mlp_head(pooled_parts, w1, b1, w2, b2):
    return pl.pallas_call(
        _mlp_head_body,
        out_shape=jax.ShapeDtypeStruct((GRAPHS, OUT_DIM), jnp.float32),
    )(pooled_parts, w1, b1[None], w2, b2[None])


def _prep_a(a):
    return a.reshape(4, 2, 32).transpose(1, 0, 2).reshape(2, 128)


def kernel(x, edge_index, edge_weight, batch, W_gcn, b_gcn, g1_W, g1_as, g1_ad, g1_b, g2_W, g2_as, g2_ad, g2_b, fc1_W, fc1_b, fc2_W, fc2_b, pro_bias, gfc1_W, gfc1_b, gfc2_W, gfc2_b):
    src = edge_index[0]
    dst = edge_index[1]
    npad = EP - E
    src = jnp.concatenate([src, jnp.zeros((npad,), jnp.int32)])
    dst = jnp.concatenate([dst, jnp.full((npad,), N, jnp.int32)])
    ew_p = jnp.concatenate([edge_weight, jnp.zeros((npad,), jnp.float32)])
    z16 = jnp.zeros((N, L), jnp.float32)
    z32 = jnp.zeros((N, 32), jnp.float32)

    def padn(a):
        return jnp.concatenate([a, jnp.zeros((8, a.shape[1]), a.dtype)])
    g128 = (jnp.arange(128)[:, None] // 32 == jnp.arange(4)[None, :]).astype(jnp.float32)
    g4 = (jnp.arange(16)[:, None] == jnp.arange(128)[None, :] // 32).astype(jnp.float32)
    r128 = (jnp.arange(128)[:, None] % 32 == jnp.arange(32)[None, :]).astype(jnp.float32)

    w16 = jnp.broadcast_to(ew_p[:, None], (EP, L))
    degp = _deg_pass(dst, w16, z16).reshape(2, N, L)
    hp2, dinv8 = _tc2(x, W_gcn, degp)
    gacc = _gcn_pass(hp2.reshape(2 * N, 32), src, dst, ew_p, z32)
    xcur, Hsc, als16, ald16 = _tc3(gacc.reshape(2, N, 32), hp2, dinv8,
                                   b_gcn[None], g1_W, _prep_a(g1_as),
                                   _prep_a(g1_ad), g128)

    zp, exb = _gat_logit_pass(padn(als16), padn(ald16), src, dst, z32)
    zinv16, selfa16 = _tc4(zp.reshape(2, N, 32), als16, ald16)
    acc1 = _gat_out_pass(Hsc.reshape(2 * N, 128), src, dst, exb, padn(zinv16), z32)
    xcur, Hsc2, als26, ald26 = _tc5_layer1(
        acc1.reshape(2, N, 32), Hsc, selfa16, xcur, g1_b[None], g4, r128,
        fc1_W, fc1_b[None], fc2_W, fc2_b[None], pro_bias[None],
        g2_W, _prep_a(g2_as), _prep_a(g2_ad), g128)

    zp2, exb2 = _gat_logit_pass(padn(als26), padn(ald26), src, dst, z32)
    zinv26, selfa26 = _tc4(zp2.reshape(2, N, 32), als26, ald26)
    acc2 = _gat_out_pass(Hsc2.reshape(2 * N, 128), src, dst, exb2, padn(zinv26), z32)
    (xcur,) = _tc5_layer2(
        acc2.reshape(2, N, 32), Hsc2, selfa26, xcur, g2_b[None], g4, r128,
        fc1_W, fc1_b[None], fc2_W, fc2_b[None], pro_bias[None])

    parts = _pool_pass(xcur, batch).reshape(NC * NT, GRAPHS, HID)
    return _mlp_head(parts, gfc1_W, gfc1_b, gfc2_W, gfc2_b)
